# Initial kernel scaffold; baseline (speedup 1.0000x reference)
import jax
import jax.numpy as jnp
from jax.experimental import pallas as pl


def kernel(pred_tensor, target_tensor):
    raise NotImplementedError("write your pallas kernel here")


# speedup vs baseline: 43.0376x; 43.0376x over previous
import functools

import jax
import jax.numpy as jnp
from jax import lax
from jax.experimental import pallas as pl
from jax.experimental.pallas import tpu as pltpu
from jax.experimental.pallas import tpu_sc as plsc

_B = 1024
_CLS = 15
_S = 32
_GRID = _S * _S
_SLAB = _CLS * _S
_NROW = _S * _SLAB
_L = 16
_NSUB = 16
_RPS = _B // _NSUB
_NG = _RPS // _L
_FBUF = 1536

_MESH = plsc.VectorSubcoreMesh(
    core_axis_name="c", subcore_axis_name="s", num_cores=2, num_subcores=_NSUB
)


@functools.partial(
    pl.kernel,
    out_type=jax.ShapeDtypeStruct((_L,), jnp.float32),
    mesh=_MESH,
    compiler_params=pltpu.CompilerParams(needs_layout_passes=False,
                                         use_tc_tiling_on_sc=True),
    scratch_types=[
        pltpu.VMEM((_SLAB, 128), jnp.float32),
        pltpu.VMEM((_FBUF,), jnp.int32),
        pltpu.VMEM((_CLS * _L, 128), jnp.float32),
        pltpu.VMEM((_CLS * _L, 128), jnp.float32),
        pltpu.VMEM((_L,), jnp.float32),
        pltpu.VMEM((_NSUB * _L,), jnp.float32),
        pltpu.VMEM((_L,), jnp.float32),
        pltpu.VMEM_SHARED((_NSUB * _L,), jnp.float32),
        pltpu.SMEM((1,), jnp.int32),
        pltpu.SemaphoreType.DMA,
        pltpu.SemaphoreType.DMA,
    ],
)
def _sc_loss(pnat, tnat, out, slab_v, fbuf_v, pgat_v, tgat_v, part_v, big_v,
             out_v, shared_ref, found_ref, sem1, sem2):
    sid = lax.axis_index("s")
    cid = lax.axis_index("c")

    def zero_body(i, carry):
        fbuf_v[pl.ds(i * _L, _L)] = jnp.zeros((_L,), jnp.int32)
        return carry

    lax.fori_loop(0, _B // _L, zero_body, jnp.int32(0))
    found_ref[0] = jnp.int32(0)

    def compact_vec(b, j, bl):
        def body(sv, fnd):
            iota = lax.iota(jnp.int32, _L)
            svec = sv * _L + iota
            cvec = svec % _CLS
            kvec = svec // _CLS
            lr = cvec * _S + kvec
            x = plsc.load_gather(slab_v, [lr, bl + jnp.zeros((_L,),
                                                             jnp.int32)])
            m = x != 0.0
            mi = m.astype(jnp.int32)
            cs = plsc.cumsum(mi)
            dest = fnd + cs - 1
            flat = b * (_GRID * _CLS) + j * _SLAB + svec
            plsc.store_scatter(fbuf_v, [dest], flat, mask=m)
            return fnd + jnp.sum(mi)
        return body

    def scan_slab(j, b):
        @pl.when(found_ref[0] < _B)
        def _():
            cb = pl.multiple_of((b // 128) * 128, 128)
            r0 = pl.multiple_of(j * _SLAB, _SLAB)
            pltpu.sync_copy(tnat.at[pl.ds(r0, _SLAB), pl.ds(cb, 128)],
                            slab_v)
            found_ref[0] = lax.fori_loop(0, _SLAB // _L,
                                         compact_vec(b, j, b % 128),
                                         found_ref[0])
        return b

    def scan_sample(b, carry):
        @pl.when(found_ref[0] < _B)
        def _():
            lax.fori_loop(0, _S, scan_slab, b)
        return carry

    lax.fori_loop(0, _B, scan_sample, jnp.int32(0))

    iota = lax.iota(jnp.int32, _L)
    base = sid * _RPS
    acc = jnp.zeros((_L,), jnp.float32)
    for g in range(_NG):
        fvec = fbuf_v[pl.ds(base + g * _L, _L)]
        cell = (fvec // _CLS) % _GRID
        r0 = (cell // _S) * _SLAB + cell % _S
        ib = base + g * _L
        cb = pl.multiple_of((ib // 128) * 128, 128)
        ll = ib - cb + iota
        copies = []
        for c in range(_CLS):
            rv = r0 + c * _S
            copies.append(pltpu.async_copy(
                pnat.at[rv, pl.ds(cb, 128)],
                pgat_v.at[pl.ds(c * _L, _L)], sem1))
            copies.append(pltpu.async_copy(
                tnat.at[rv, pl.ds(cb, 128)],
                tgat_v.at[pl.ds(c * _L, _L)], sem2))
        for cp in copies:
            cp.wait()

        ps = [plsc.load_gather(pgat_v, [c * _L + iota, ll])
              for c in range(_CLS)]
        m = ps[0]
        for c in range(1, _CLS):
            m = jnp.maximum(m, ps[c])
        es = [jnp.exp(p - m) for p in ps]
        s = es[0]
        for c in range(1, _CLS):
            s = s + es[c]
        inv = 1.0 / s
        for c in range(_CLS):
            t = plsc.load_gather(tgat_v, [c * _L + iota, ll])
            d = es[c] * inv - t
            acc = acc + d * d

    part_v[...] = acc
    pltpu.sync_copy(part_v,
                    shared_ref.at[pl.ds(pl.multiple_of(sid * _L, _L), _L)])
    plsc.subcore_barrier()

    @pl.when(jnp.logical_and(sid == 0, cid == 0))
    def _():
        pltpu.sync_copy(shared_ref, big_v)
        tot = jnp.zeros((_L,), jnp.float32)
        for r in range(_NSUB):
            tot = tot + big_v[pl.ds(r * _L, _L)]
        loss = jnp.sum(tot) * (1.0 / _B)
        out_v[...] = jnp.broadcast_to(loss, (_L,))
        pltpu.sync_copy(out_v, out)


def kernel(pred_tensor, target_tensor):
    pnat = pred_tensor.transpose(1, 3, 2, 0).reshape(_NROW, _B)
    tnat = target_tensor.transpose(1, 3, 2, 0).reshape(_NROW, _B)
    return _sc_loss(pnat, tnat)[0]


# speedup vs baseline: 59.8906x; 1.3916x over previous
import functools

import jax
import jax.numpy as jnp
from jax import lax
from jax.experimental import pallas as pl
from jax.experimental.pallas import tpu as pltpu
from jax.experimental.pallas import tpu_sc as plsc

_B = 1024
_CLS = 15
_S = 32
_GRID = _S * _S
_SLAB = _CLS * _S
_NROW = _S * _SLAB
_L = 16
_NSUB = 16
_RPS = _B // _NSUB
_NG = _RPS // _L
_JG = 4
_SS = _JG * _SLAB
_RPT = _SS // 15
_NSS = _S // _JG
_FBUF = 3072

_MESH = plsc.VectorSubcoreMesh(
    core_axis_name="c", subcore_axis_name="s", num_cores=2, num_subcores=_NSUB
)


@functools.partial(
    pl.kernel,
    out_type=jax.ShapeDtypeStruct((_L,), jnp.float32),
    mesh=_MESH,
    compiler_params=pltpu.CompilerParams(needs_layout_passes=False,
                                         use_tc_tiling_on_sc=True),
    scratch_types=[
        pltpu.VMEM((_RPT, 128), jnp.float32),
        pltpu.VMEM((_RPT,), jnp.float32),
        pltpu.VMEM((_SS,), jnp.float32),
        pltpu.VMEM_SHARED((_SS,), jnp.float32),
        pltpu.VMEM((_FBUF,), jnp.int32),
        pltpu.VMEM((_CLS * _L, 128), jnp.float32),
        pltpu.VMEM((_CLS * _L, 128), jnp.float32),
        pltpu.VMEM((_L,), jnp.float32),
        pltpu.VMEM((_NSUB * _L,), jnp.float32),
        pltpu.VMEM((_L,), jnp.float32),
        pltpu.VMEM_SHARED((_NSUB * _L,), jnp.float32),
        pltpu.SMEM((1,), jnp.int32),
        pltpu.SemaphoreType.DMA,
        pltpu.SemaphoreType.DMA,
    ],
)
def _sc_loss(pnat, tnat, out, mini_v, stage_v, sval_v, sshare, fbuf_v,
             pgat_v, tgat_v, part_v, big_v, out_v, shared_ref, found_ref,
             sem1, sem2):
    sid = lax.axis_index("s")
    cid = lax.axis_index("c")

    def zero_body(i, carry):
        fbuf_v[pl.ds(i * _L, _L)] = jnp.zeros((_L,), jnp.int32)
        return carry

    lax.fori_loop(0, _B // _L, zero_body, jnp.int32(0))
    found_ref[0] = jnp.int32(0)

    def compact_vec(b, jg):
        def body(sv, fnd):
            iota = lax.iota(jnp.int32, _L)
            svec = sv * _L + iota
            jloc = svec // _SLAB
            srem = svec % _SLAB
            lr = jloc * _SLAB + (srem % _CLS) * _S + srem // _CLS
            x = plsc.load_gather(sval_v, [lr])
            m = x != 0.0
            mi = m.astype(jnp.int32)
            cs = plsc.cumsum(mi)
            dest = fnd + cs - 1
            flat = b * (_GRID * _CLS) + jg * _SS + svec
            plsc.store_scatter(fbuf_v, [dest], flat, mask=m)
            return fnd + cs[_L - 1]
        return body

    def scan_superslab(b, jg):
        @pl.when(sid < 15)
        def _():
            cb = pl.multiple_of((b // 128) * 128, 128)
            myr = pl.multiple_of(jg * _SS + sid * _RPT, _RPT)
            pltpu.sync_copy(tnat.at[pl.ds(myr, _RPT), pl.ds(cb, 128)],
                            mini_v)
            bl = b % 128
            iota = lax.iota(jnp.int32, _L)
            for v in range(_RPT // _L):
                stage_v[pl.ds(v * _L, _L)] = plsc.load_gather(
                    mini_v, [v * _L + iota, bl + jnp.zeros((_L,),
                                                           jnp.int32)])
            pltpu.sync_copy(
                stage_v, sshare.at[pl.ds(pl.multiple_of(sid * _RPT, _RPT),
                                         _RPT)])
        plsc.subcore_barrier()
        pltpu.sync_copy(sshare, sval_v)
        plsc.subcore_barrier()
        found_ref[0] = lax.fori_loop(0, _SS // _L, compact_vec(b, jg),
                                     found_ref[0])

    scan_superslab(jnp.int32(0), jnp.int32(0))

    @pl.when(found_ref[0] < _B)
    def _():
        def rest_of_sample0(jg, carry):
            @pl.when(found_ref[0] < _B)
            def _():
                scan_superslab(jnp.int32(0), jg)
            return carry

        lax.fori_loop(1, _NSS, rest_of_sample0, jnp.int32(0))

        def scan_sample(b, carry):
            @pl.when(found_ref[0] < _B)
            def _():
                def ss_body(jg, bb):
                    @pl.when(found_ref[0] < _B)
                    def _():
                        scan_superslab(bb, jg)
                    return bb
                lax.fori_loop(0, _NSS, ss_body, b)
            return carry

        lax.fori_loop(1, _B, scan_sample, jnp.int32(0))

    iota = lax.iota(jnp.int32, _L)
    base = sid * _RPS
    acc = jnp.zeros((_L,), jnp.float32)
    for g in range(_NG):
        fvec = fbuf_v[pl.ds(base + g * _L, _L)]
        cell = (fvec // _CLS) % _GRID
        r0 = (cell // _S) * _SLAB + cell % _S
        ib = base + g * _L
        cb = pl.multiple_of((ib // 128) * 128, 128)
        ll = ib - cb + iota
        copies = []
        for c in range(_CLS):
            rv = r0 + c * _S
            copies.append(pltpu.async_copy(
                pnat.at[rv, pl.ds(cb, 128)],
                pgat_v.at[pl.ds(c * _L, _L)], sem1))
            copies.append(pltpu.async_copy(
                tnat.at[rv, pl.ds(cb, 128)],
                tgat_v.at[pl.ds(c * _L, _L)], sem2))
        for cp in copies:
            cp.wait()

        ps = [plsc.load_gather(pgat_v, [c * _L + iota, ll])
              for c in range(_CLS)]
        m = ps[0]
        for c in range(1, _CLS):
            m = jnp.maximum(m, ps[c])
        es = [jnp.exp(p - m) for p in ps]
        s = es[0]
        for c in range(1, _CLS):
            s = s + es[c]
        inv = 1.0 / s
        for c in range(_CLS):
            t = plsc.load_gather(tgat_v, [c * _L + iota, ll])
            d = es[c] * inv - t
            acc = acc + d * d

    part_v[...] = acc
    pltpu.sync_copy(part_v,
                    shared_ref.at[pl.ds(pl.multiple_of(sid * _L, _L), _L)])
    plsc.subcore_barrier()

    @pl.when(jnp.logical_and(sid == 0, cid == 0))
    def _():
        pltpu.sync_copy(shared_ref, big_v)
        tot = jnp.zeros((_L,), jnp.float32)
        for r in range(_NSUB):
            tot = tot + big_v[pl.ds(r * _L, _L)]
        loss = jnp.sum(tot) * (1.0 / _B)
        out_v[...] = jnp.broadcast_to(loss, (_L,))
        pltpu.sync_copy(out_v, out)


def kernel(pred_tensor, target_tensor):
    pnat = pred_tensor.transpose(1, 3, 2, 0).reshape(_NROW, _B)
    tnat = target_tensor.transpose(1, 3, 2, 0).reshape(_NROW, _B)
    return _sc_loss(pnat, tnat)[0]
